# baseline (speedup 1.0000x reference)
import dataclasses
import functools

import jax
import jax.numpy as jnp
from jax import lax
from jax.experimental import pallas as pl
from jax.experimental.pallas import tpu as pltpu
from jax.experimental.pallas import tpu_sc as plsc

_NUM_OPTIONS = 16
_NUM_ACTIONS = 32
_D_SMALL = _NUM_OPTIONS
_D_IOP = _NUM_OPTIONS * _NUM_ACTIONS
_NC, _NS = 2, 16
_NW = _NC * _NS
_L = 16
_CH = 32
_NCHUNK = 4

_mesh = plsc.VectorSubcoreMesh(core_axis_name="c", subcore_axis_name="s")


def _compiler_params():
    cp = pltpu.CompilerParams()
    if "needs_layout_passes" in pltpu.CompilerParams.__dataclass_fields__:
        cp = dataclasses.replace(cp, needs_layout_passes=False)
    return cp


_AHALF = _NUM_ACTIONS // 2
_D_HALF = _NUM_OPTIONS * _AHALF


@functools.lru_cache(maxsize=None)
def _build_iop_half(B, V):
    bpw = B // _NW
    assert bpw == _CH * _NCHUNK

    def body(obs_hbm, iop_hbm, out2,
             idx4, buf0, buf1, stage, sem_a, sem_b):
        wid = lax.axis_index("s") * _NC + lax.axis_index("c")
        base = wid * bpw
        for c in range(_NCHUNK):
            pltpu.sync_copy(obs_hbm.at[pl.ds(base + c * _CH, _CH)], idx4.at[c])

        bufs = (buf0, buf1)
        sems = (sem_a, sem_b)

        def fire(c):
            return pltpu.async_copy(
                iop_hbm.at[idx4.at[c]], bufs[c % 2], sems[c % 2])

        lane = lax.iota(jnp.int32, _L)

        def transpose_chunk(c, buf):
            @pl.loop(0, _CH)
            def _(bl):
                col0 = (c * _CH + bl) * _NUM_OPTIONS
                for a in range(_AHALF):
                    bl_b = jnp.full((_L,), bl, jnp.int32)
                    v = plsc.load_gather(buf, [bl_b, lane * _AHALF + a])
                    stage[a, pl.ds(col0, _NUM_OPTIONS)] = v

        g0 = fire(0)
        g1 = fire(1)
        g0.wait()
        transpose_chunk(0, buf0)
        g2 = fire(2)
        g1.wait()
        transpose_chunk(1, buf1)
        g3 = fire(3)
        g2.wait()
        transpose_chunk(2, buf0)
        g3.wait()
        transpose_chunk(3, buf1)
        pltpu.sync_copy(
            stage,
            out2.at[:, pl.ds(wid * (bpw * _NUM_OPTIONS), bpw * _NUM_OPTIONS)],
        )

    return pl.kernel(
        body,
        compiler_params=_compiler_params(),
        out_type=[
            jax.ShapeDtypeStruct((_AHALF, B * _NUM_OPTIONS), jnp.float32)
        ],
        mesh=_mesh,
        scratch_types=[
            pltpu.VMEM((_NCHUNK, _CH), jnp.int32),
            pltpu.VMEM((_CH, _D_HALF), jnp.float32),
            pltpu.VMEM((_CH, _D_HALF), jnp.float32),
            pltpu.VMEM((_AHALF, (B // _NW) * _NUM_OPTIONS), jnp.float32),
            pltpu.SemaphoreType.DMA,
            pltpu.SemaphoreType.DMA,
        ],
    )


@functools.lru_cache(maxsize=None)
def _build_small(B, V):
    bpw = B // _NW
    wide_pw = bpw * _D_SMALL // 128

    def body(obs_hbm, betaT, pooT, qT,
             beta_o, poo_o, q_o,
             idx_v, sb0, sb1, beta_c, poo_c, q_c,
             sem_a, sem_b):
        wid = lax.axis_index("s") * _NC + lax.axis_index("c")
        base = wid * bpw
        pltpu.sync_copy(obs_hbm.at[pl.ds(base, bpw)], idx_v)

        sbufs = (sb0, sb1)
        sems = (sem_a, sem_b)
        lane = lax.iota(jnp.int32, _L)

        def fire(o, par):
            cb = pl.multiple_of(lax.bitwise_and(o, jnp.int32(~127)), 128)
            sb = sbufs[par]
            pltpu.async_copy(betaT.at[:, pl.ds(cb, 128)], sb.at[0], sems[par])
            pltpu.async_copy(pooT.at[:, pl.ds(cb, 128)], sb.at[1], sems[par])
            pltpu.async_copy(qT.at[:, pl.ds(cb, 128)], sb.at[2], sems[par])

        def drain(par):
            for t in range(3):
                pltpu.make_async_copy(
                    betaT.at[:, pl.ds(0, 128)], sbufs[par].at[t],
                    sems[par]).wait()

        def process(o, i, par):
            om = lax.bitwise_and(o, 127)
            r8 = lax.shift_right_logical(i, 3)
            c0 = lax.bitwise_and(i, 7) * _D_SMALL
            sb = sbufs[par]
            om_b = jnp.full((_L,), om, jnp.int32)
            bv = plsc.load_gather(sb.at[0], [lane, om_b])
            beta_c[r8, pl.ds(c0, _D_SMALL)] = 1.0 / (1.0 + jnp.exp(-bv))
            poo_c[r8, pl.ds(c0, _D_SMALL)] = plsc.load_gather(
                sb.at[1], [lane, om_b])
            q_c[r8, pl.ds(c0, _D_SMALL)] = plsc.load_gather(
                sb.at[2], [lane, om_b])

        ov0 = idx_v[pl.ds(0, _L)]
        fire(ov0[0], 0)
        fire(ov0[1], 1)

        @pl.loop(0, bpw // _L)
        def _(m):
            ov = idx_v[pl.ds(m * _L, _L)]
            nxt = jnp.minimum((m + 1) * _L, bpw - _L)
            ovn = idx_v[pl.ds(nxt, _L)]
            last = m == bpw // _L - 1
            for l in range(_L):
                p = l % 2
                drain(p)
                process(ov[l], m * _L + l, p)
                if l < _L - 2:
                    fire(ov[l + 2], p)
                else:
                    @pl.when(jnp.logical_not(last))
                    def _():
                        fire(ovn[l - (_L - 2)], p)

        pltpu.sync_copy(beta_c, beta_o.at[pl.ds(wid * wide_pw, wide_pw)])
        pltpu.sync_copy(poo_c, poo_o.at[pl.ds(wid * wide_pw, wide_pw)])
        pltpu.sync_copy(q_c, q_o.at[pl.ds(wid * wide_pw, wide_pw)])

    return pl.kernel(
        body,
        compiler_params=_compiler_params(),
        out_type=[
            jax.ShapeDtypeStruct((B * _D_SMALL // 128, 128), jnp.float32),
            jax.ShapeDtypeStruct((B * _D_SMALL // 128, 128), jnp.float32),
            jax.ShapeDtypeStruct((B * _D_SMALL // 128, 128), jnp.float32),
        ],
        mesh=_mesh,
        scratch_types=[
            pltpu.VMEM((bpw,), jnp.int32),
            pltpu.VMEM((3, _D_SMALL, 128), jnp.float32),
            pltpu.VMEM((3, _D_SMALL, 128), jnp.float32),
            pltpu.VMEM((bpw * _D_SMALL // 128, 128), jnp.float32),
            pltpu.VMEM((bpw * _D_SMALL // 128, 128), jnp.float32),
            pltpu.VMEM((bpw * _D_SMALL // 128, 128), jnp.float32),
            pltpu.SemaphoreType.DMA,
            pltpu.SemaphoreType.DMA,
        ],
    )


@jax.jit
def kernel(obs, beta, iop, poo, q):
    B = obs.shape[0]
    V = iop.shape[0]
    iop_a = iop[:, :, :_AHALF].reshape(V, _D_HALF)
    iop_b = iop[:, :, _AHALF:].reshape(V, _D_HALF)
    (out_a,) = _build_iop_half(B, V)(obs, iop_a)
    (out_b,) = _build_iop_half(B, V)(obs, iop_b)
    beta_o, poo_o, q_o = _build_small(B, V)(obs, beta.T, poo.T, q.T)
    out2 = jnp.concatenate([out_a, out_b], axis=0)
    return (
        beta_o.reshape(-1),
        jnp.transpose(out2),
        poo_o.reshape(-1),
        q_o.reshape(-1),
    )


# speedup vs baseline: 1.5916x; 1.5916x over previous
import dataclasses
import functools

import jax
import jax.numpy as jnp
from jax import lax
from jax.experimental import pallas as pl
from jax.experimental.pallas import tpu as pltpu
from jax.experimental.pallas import tpu_sc as plsc

_NUM_OPTIONS = 16
_NUM_ACTIONS = 32
_D_SMALL = _NUM_OPTIONS
_D_IOP = _NUM_OPTIONS * _NUM_ACTIONS
_NC, _NS = 2, 16
_NW = _NC * _NS
_L = 16
_CH = 32
_NCHUNK = 4

_mesh = plsc.VectorSubcoreMesh(core_axis_name="c", subcore_axis_name="s")


def _compiler_params():
    cp = pltpu.CompilerParams()
    if "needs_layout_passes" in pltpu.CompilerParams.__dataclass_fields__:
        cp = dataclasses.replace(cp, needs_layout_passes=False)
    return cp


@functools.lru_cache(maxsize=None)
def _build_iop(B, V):
    bpw = B // _NW
    assert bpw == _CH * _NCHUNK

    def body(obs_hbm, iop_hbm, out2,
             idx4, buf0, buf1, stage, sem_a, sem_b):
        wid = lax.axis_index("s") * _NC + lax.axis_index("c")
        base = wid * bpw
        for c in range(_NCHUNK):
            pltpu.sync_copy(obs_hbm.at[pl.ds(base + c * _CH, _CH)], idx4.at[c])

        bufs = (buf0, buf1)
        sems = (sem_a, sem_b)

        def fire(c):
            return pltpu.async_copy(
                iop_hbm.at[idx4.at[c]], bufs[c % 2], sems[c % 2])

        lane = lax.iota(jnp.int32, _L)

        def transpose_chunk(c, buf):
            @pl.loop(0, _CH)
            def _(bl):
                col0 = (c * _CH + bl) * _NUM_OPTIONS
                for a in range(_NUM_ACTIONS):
                    bl_b = jnp.full((_L,), bl, jnp.int32)
                    v = plsc.load_gather(buf, [bl_b, lane * _NUM_ACTIONS + a])
                    stage[a, pl.ds(col0, _NUM_OPTIONS)] = v

        g0 = fire(0)
        g1 = fire(1)
        g0.wait()
        transpose_chunk(0, buf0)
        g2 = fire(2)
        g1.wait()
        transpose_chunk(1, buf1)
        g3 = fire(3)
        g2.wait()
        transpose_chunk(2, buf0)
        g3.wait()
        transpose_chunk(3, buf1)
        pltpu.sync_copy(
            stage,
            out2.at[:, pl.ds(wid * (bpw * _NUM_OPTIONS), bpw * _NUM_OPTIONS)],
        )

    return pl.kernel(
        body,
        compiler_params=_compiler_params(),
        out_type=[
            jax.ShapeDtypeStruct((_NUM_ACTIONS, B * _NUM_OPTIONS), jnp.float32)
        ],
        mesh=_mesh,
        scratch_types=[
            pltpu.VMEM((_NCHUNK, _CH), jnp.int32),
            pltpu.VMEM((_CH, _D_IOP), jnp.float32),
            pltpu.VMEM((_CH, _D_IOP), jnp.float32),
            pltpu.VMEM((_NUM_ACTIONS, (B // _NW) * _NUM_OPTIONS), jnp.float32),
            pltpu.SemaphoreType.DMA,
            pltpu.SemaphoreType.DMA,
        ],
    )


@functools.lru_cache(maxsize=None)
def _build_small(B, V):
    bpw = B // _NW
    wide_pw = bpw * _D_SMALL // 128

    def body(obs_hbm, betaT, pooT, qT,
             beta_o, poo_o, q_o,
             idx_v, sb0, sb1, beta_c, poo_c, q_c,
             sem_a, sem_b):
        wid = lax.axis_index("s") * _NC + lax.axis_index("c")
        base = wid * bpw
        pltpu.sync_copy(obs_hbm.at[pl.ds(base, bpw)], idx_v)

        sbufs = (sb0, sb1)
        sems = (sem_a, sem_b)
        lane = lax.iota(jnp.int32, _L)

        def fire(o, par):
            cb = pl.multiple_of(lax.bitwise_and(o, jnp.int32(~127)), 128)
            sb = sbufs[par]
            pltpu.async_copy(betaT.at[:, pl.ds(cb, 128)], sb.at[0], sems[par])
            pltpu.async_copy(pooT.at[:, pl.ds(cb, 128)], sb.at[1], sems[par])
            pltpu.async_copy(qT.at[:, pl.ds(cb, 128)], sb.at[2], sems[par])

        def drain(par):
            for t in range(3):
                pltpu.make_async_copy(
                    betaT.at[:, pl.ds(0, 128)], sbufs[par].at[t],
                    sems[par]).wait()

        def process(o, i, par):
            om = lax.bitwise_and(o, 127)
            r8 = lax.shift_right_logical(i, 3)
            c0 = lax.bitwise_and(i, 7) * _D_SMALL
            sb = sbufs[par]
            om_b = jnp.full((_L,), om, jnp.int32)
            bv = plsc.load_gather(sb.at[0], [lane, om_b])
            beta_c[r8, pl.ds(c0, _D_SMALL)] = 1.0 / (1.0 + jnp.exp(-bv))
            poo_c[r8, pl.ds(c0, _D_SMALL)] = plsc.load_gather(
                sb.at[1], [lane, om_b])
            q_c[r8, pl.ds(c0, _D_SMALL)] = plsc.load_gather(
                sb.at[2], [lane, om_b])

        ov0 = idx_v[pl.ds(0, _L)]
        fire(ov0[0], 0)
        fire(ov0[1], 1)

        @pl.loop(0, bpw // _L)
        def _(m):
            ov = idx_v[pl.ds(m * _L, _L)]
            nxt = jnp.minimum((m + 1) * _L, bpw - _L)
            ovn = idx_v[pl.ds(nxt, _L)]
            last = m == bpw // _L - 1
            for l in range(_L):
                p = l % 2
                drain(p)
                process(ov[l], m * _L + l, p)
                if l < _L - 2:
                    fire(ov[l + 2], p)
                else:
                    @pl.when(jnp.logical_not(last))
                    def _():
                        fire(ovn[l - (_L - 2)], p)

        pltpu.sync_copy(beta_c, beta_o.at[pl.ds(wid * wide_pw, wide_pw)])
        pltpu.sync_copy(poo_c, poo_o.at[pl.ds(wid * wide_pw, wide_pw)])
        pltpu.sync_copy(q_c, q_o.at[pl.ds(wid * wide_pw, wide_pw)])

    return pl.kernel(
        body,
        compiler_params=_compiler_params(),
        out_type=[
            jax.ShapeDtypeStruct((B * _D_SMALL // 128, 128), jnp.float32),
            jax.ShapeDtypeStruct((B * _D_SMALL // 128, 128), jnp.float32),
            jax.ShapeDtypeStruct((B * _D_SMALL // 128, 128), jnp.float32),
        ],
        mesh=_mesh,
        scratch_types=[
            pltpu.VMEM((bpw,), jnp.int32),
            pltpu.VMEM((3, _D_SMALL, 128), jnp.float32),
            pltpu.VMEM((3, _D_SMALL, 128), jnp.float32),
            pltpu.VMEM((bpw * _D_SMALL // 128, 128), jnp.float32),
            pltpu.VMEM((bpw * _D_SMALL // 128, 128), jnp.float32),
            pltpu.VMEM((bpw * _D_SMALL // 128, 128), jnp.float32),
            pltpu.SemaphoreType.DMA,
            pltpu.SemaphoreType.DMA,
        ],
    )


@jax.jit
def kernel(obs, beta, iop, poo, q):
    B = obs.shape[0]
    V = iop.shape[0]
    iop2 = iop.reshape(V, _D_IOP)
    (out2,) = _build_iop(B, V)(obs, iop2)
    beta_o, poo_o, q_o = _build_small(B, V)(obs, beta.T, poo.T, q.T)
    return (
        beta_o.reshape(-1),
        jnp.transpose(out2),
        poo_o.reshape(-1),
        q_o.reshape(-1),
    )
